# baseline (speedup 1.0000x reference)
import functools

import jax
import jax.numpy as jnp
from jax import lax
from jax.experimental import pallas as pl
from jax.experimental.pallas import tpu as pltpu
from jax.experimental.pallas import tpu_sc as plsc

B = 128
N = 32768
LANES = 16
G = 16
CB = 256
U = 8
CHUNK = 32
CHUNKS = 128 // CHUNK
BIG = 2**30


def _merge(m_a, p_a, m_b, p_b):
    better = (m_b > m_a) | ((m_b == m_a) & (p_b < p_a))
    return jnp.where(better, m_b, m_a), jnp.where(better, p_b, p_a)


@functools.partial(
    pl.kernel,
    out_type=(jax.ShapeDtypeStruct((G, CB, 8, 128), jnp.float32),
              jax.ShapeDtypeStruct((512,), jnp.float32),
              jax.ShapeDtypeStruct((512,), jnp.int32)),
    mesh=plsc.VectorSubcoreMesh(core_axis_name="c", subcore_axis_name="s"),
    compiler_params=pltpu.CompilerParams(needs_layout_passes=False),
    scratch_types=[
        pltpu.VMEM((CHUNK, 8, 128), jnp.float32),
        pltpu.VMEM((CHUNK, 8, 128), jnp.float32),
        pltpu.VMEM((8, 8, 128), jnp.float32),
        pltpu.VMEM((LANES,), jnp.float32),
        pltpu.VMEM((LANES,), jnp.int32),
        pltpu.VMEM((LANES,), jnp.float32),
        pltpu.VMEM((LANES,), jnp.int32),
        pltpu.VMEM((LANES,), jnp.int32),
        pltpu.VMEM((8, LANES), jnp.float32),
        pltpu.SemaphoreType.DMA((2,)),
        pltpu.SemaphoreType.DMA,
        pltpu.SemaphoreType.DMA,
    ],
)
def _one_hot_argmax(y_hbm, out_hbm, m_hbm, p_hbm, buf0, buf1, zbuf, mym, myp,
                    nbm, nbp, fpbuf, patch, sem_in, sem_z, sem_f):
    cid = lax.axis_index("c")
    sid = lax.axis_index("s")
    g = cid * 8 + sid // 2
    half = sid % 2
    c0 = half * 128

    bufs = [buf0, buf1]
    handles = [None] * CHUNKS
    handles[0] = pltpu.async_copy(y_hbm.at[g, pl.ds(c0, CHUNK)], bufs[0],
                                  sem_in.at[0])

    zeros16 = jnp.zeros((LANES,), jnp.float32)

    def zfill(t, carry):
        for s_ in range(8):
            for k in range(8):
                zbuf[t, s_, pl.ds(k * LANES, LANES)] = zeros16
        return carry

    lax.fori_loop(0, 8, zfill, 0)

    zh = [
        pltpu.async_copy(zbuf, out_hbm.at[g, pl.ds(c0 + j * 8, 8)], sem_z)
        for j in range(128 // 8)
    ]

    lane = lax.broadcasted_iota(jnp.int32, (LANES,), 0)
    neg_inf = jnp.full((LANES,), -jnp.inf, jnp.float32)
    zero_i = jnp.zeros((LANES,), jnp.int32)
    run = [(neg_inf, zero_i)] * 8

    for ch in range(CHUNKS):
        handles[ch].wait()
        if ch + 1 < CHUNKS:
            handles[ch + 1] = pltpu.async_copy(
                y_hbm.at[g, pl.ds(c0 + (ch + 1) * CHUNK, CHUNK)],
                bufs[(ch + 1) % 2], sem_in.at[(ch + 1) % 2])
        buf = bufs[ch % 2]

        def body(t, carry, _buf=buf):
            ibase, maxs, iters = carry
            maxs, iters = list(maxs), list(iters)
            for k in range(8):
                idxk = ibase + k
                for s in range(8):
                    v = _buf[t, s, pl.ds(k * LANES, LANES)]
                    pred = v > maxs[s]
                    maxs[s] = jnp.where(pred, v, maxs[s])
                    iters[s] = jnp.where(pred, idxk, iters[s])
            return ibase + 8, tuple(maxs), tuple(iters)

        _, maxs, iters = lax.fori_loop(
            0, CHUNK, body, (zero_i, (neg_inf,) * 8, (zero_i,) * 8))

        base_off = (c0 + ch * CHUNK) * 128
        for s in range(8):
            ti = iters[s] // 8
            pu = base_off + ti * 128 + (iters[s] - ti * 8) * LANES + lane
            run[s] = _merge(run[s][0], run[s][1], maxs[s], pu)

    mvec = neg_inf
    pvec = zero_i
    for s in range(8):
        rm, rp = run[s]
        m = jnp.max(rm)
        p = jnp.min(jnp.where(rm == m, rp, BIG))
        mvec = jnp.where(lane == s, m, mvec)
        pvec = jnp.where(lane == s, p, pvec)
    mym[...] = mvec
    myp[...] = pvec

    for h in zh:
        h.wait()

    widx = cid * 16 + sid
    pltpu.sync_copy(mym, m_hbm.at[pl.ds(widx * LANES, LANES)])
    pltpu.sync_copy(myp, p_hbm.at[pl.ds(widx * LANES, LANES)])
    plsc.subcore_barrier()

    nbw = cid * 16 + sid + 1 - 2 * half
    pltpu.sync_copy(m_hbm.at[pl.ds(nbw * LANES, LANES)], nbm)
    pltpu.sync_copy(p_hbm.at[pl.ds(nbw * LANES, LANES)], nbp)
    nm = nbm[...]
    np_ = nbp[...]
    take_nb = jnp.where(half == 0, nm > mvec, nm >= mvec)
    fp = jnp.where(take_nb, np_, pvec)
    fixups = []
    for s in range(8):
        p = fp[s]
        cblk = p // 128
        l0 = ((p - cblk * 128) // LANES) * LANES
        patch[s] = jnp.where(lane == p - cblk * 128 - l0, 1.0,
                             0.0).astype(jnp.float32)
        fixups.append(
            pltpu.async_copy(patch.at[s],
                             out_hbm.at[g, cblk, s, pl.ds(l0, LANES)],
                             sem_f))
    for h in fixups:
        h.wait()


def kernel(x):
    y = x.reshape(G, 8, CB, 128).transpose(0, 2, 1, 3)
    out_y, _, _ = _one_hot_argmax(y)
    return out_y.transpose(0, 2, 1, 3).reshape(B, N)


# speedup vs baseline: 1.3262x; 1.3262x over previous
import functools

import jax
import jax.numpy as jnp
from jax import lax
from jax.experimental import pallas as pl
from jax.experimental.pallas import tpu as pltpu
from jax.experimental.pallas import tpu_sc as plsc

B = 128
N = 32768
LANES = 16
NUM_WORKERS = 32
ROWS_PER_W = B // NUM_WORKERS
U = 8
ITERS = N // (LANES * U)


def _row_argmax(buf):
    lane = lax.broadcasted_iota(jnp.int32, (LANES,), 0)
    neg_inf = jnp.full((LANES,), -jnp.inf, jnp.float32)
    zero_i = jnp.zeros((LANES,), jnp.int32)

    def body(i, carry):
        ivec, maxs, iters = carry
        maxs, iters = list(maxs), list(iters)
        base = i * (U * LANES)
        for u in range(U):
            v = buf[pl.ds(base + u * LANES, LANES)]
            pred = v > maxs[u]
            maxs[u] = jnp.where(pred, v, maxs[u])
            iters[u] = jnp.where(pred, ivec, iters[u])
        return ivec + 1, tuple(maxs), tuple(iters)

    _, maxs, iters = lax.fori_loop(
        0, ITERS, body, (zero_i, (neg_inf,) * U, (zero_i,) * U))

    best_m = maxs[0]
    best_p = (iters[0] * U + 0) * LANES + lane
    for u in range(1, U):
        p = (iters[u] * U + u) * LANES + lane
        better = (maxs[u] > best_m) | ((maxs[u] == best_m) & (p < best_p))
        best_m = jnp.where(better, maxs[u], best_m)
        best_p = jnp.where(better, p, best_p)
    m = jnp.max(best_m)
    cand = jnp.where(best_m == m, best_p, jnp.int32(2**30))
    return jnp.min(cand)


@functools.partial(
    pl.kernel,
    out_type=jax.ShapeDtypeStruct((B, N), jnp.float32),
    mesh=plsc.VectorSubcoreMesh(core_axis_name="c", subcore_axis_name="s"),
    compiler_params=pltpu.CompilerParams(needs_layout_passes=False),
    scratch_types=[
        pltpu.VMEM((N,), jnp.float32),
        pltpu.VMEM((N,), jnp.float32),
        pltpu.VMEM((N,), jnp.float32),
        pltpu.VMEM((ROWS_PER_W, LANES), jnp.float32),
        pltpu.SemaphoreType.DMA,
        pltpu.SemaphoreType.DMA((ROWS_PER_W,)),
        pltpu.SemaphoreType.DMA,
    ],
)
def _one_hot_argmax(x_hbm, out_hbm, buf0, buf1, zbuf, fixbuf, sem_in, sem_z,
                    sem_f):
    wid = lax.axis_index("s") * 2 + lax.axis_index("c")
    row0 = wid * ROWS_PER_W

    bufs = [buf0, buf1]
    handles = [None] * ROWS_PER_W
    handles[0] = pltpu.async_copy(x_hbm.at[row0], bufs[0], sem_in)

    zeros16 = jnp.zeros((LANES,), jnp.float32)

    def zfill(i, carry):
        base = i * (U * LANES)
        for u in range(U):
            zbuf[pl.ds(base + u * LANES, LANES)] = zeros16
        return carry

    lax.fori_loop(0, ITERS, zfill, 0)

    zh = [
        pltpu.async_copy(zbuf, out_hbm.at[row0 + r], sem_z.at[r])
        for r in range(ROWS_PER_W)
    ]

    lane = lax.broadcasted_iota(jnp.int32, (LANES,), 0)
    bases = []
    for r in range(ROWS_PER_W):
        handles[r].wait()
        if r + 1 < ROWS_PER_W:
            handles[r + 1] = pltpu.async_copy(x_hbm.at[row0 + r + 1],
                                              bufs[(r + 1) % 2], sem_in)
        pos = _row_argmax(bufs[r % 2])
        base = (pos // LANES) * LANES
        fixbuf[r] = jnp.where(lane == pos - base, 1.0, 0.0).astype(jnp.float32)
        bases.append(base)

    fixups = []
    for r in range(ROWS_PER_W):
        zh[r].wait()
        fixups.append(
            pltpu.async_copy(fixbuf.at[r],
                             out_hbm.at[row0 + r, pl.ds(bases[r], LANES)],
                             sem_f))
    for h in fixups:
        h.wait()


def kernel(x):
    return _one_hot_argmax(x)
